# Initial kernel scaffold; baseline (speedup 1.0000x reference)
import jax
import jax.numpy as jnp
from jax.experimental import pallas as pl


def kernel(autoregressive_embedding, unit_type_mask, entity_embedding, entity_mask, selected_units, key_fc_w, key_fc_b, func_fc_w, func_fc_b, fc1_w, fc1_b, fc2_w, fc2_b, embed_fc_w, embed_fc_b, end_embedding, lstm_w_ih, lstm_w_hh, lstm_b_ih, lstm_b_hh):
    raise NotImplementedError("write your pallas kernel here")


# speedup vs baseline: 6.0947x; 6.0947x over previous
import functools

import jax
import jax.numpy as jnp
from jax import lax
from jax.experimental import pallas as pl
from jax.experimental.pallas import tpu as pltpu

B = 64
N = 512
T = 16
KEY = 32
HID = 32
FUNC = 256
IN = 1024
UT = 259
NBLK = 64
GRID = N // NBLK


def _fused_kernel(ar_ref, utm_ref, ent_ref, sel_ref, kw_ref, kb_ref,
                  fw_ref, fb_ref, f1w_ref, f1b_ref, f2w_ref, f2b_ref,
                  ew_ref, eb_ref, endw_ref, wih_ref, whh_ref, bih_ref,
                  bhh_ref, out_ref, key_s):
    i = pl.program_id(0)

    ent2 = ent_ref[...].reshape(B * NBLK, -1)
    kblk = lax.dot_general(ent2, kw_ref[...], (((1,), (1,)), ((), ())),
                           preferred_element_type=jnp.float32)
    kblk = kblk + kb_ref[...][None, :]
    key_s[:, pl.ds(i * NBLK, NBLK), :] = kblk.reshape(B, NBLK, KEY)

    @pl.when(i == GRID - 1)
    def _decode():
        key = key_s[...]
        sel = sel_ref[...]
        iota_n = lax.broadcasted_iota(jnp.int32, (B, T, N), 2)
        oh = (sel[:, :, None] == iota_n).astype(jnp.float32)

        e_sel = lax.dot_general(oh, key, (((2,), (1,)), ((0,), (0,))),
                                preferred_element_type=jnp.float32)
        e_sel = e_sel * (1.0 / N)

        m_fused = lax.dot_general(ew_ref[...], f1w_ref[...],
                                  (((0,), (1,)), ((), ())),
                                  preferred_element_type=jnp.float32)
        cb = lax.dot_general(eb_ref[...].reshape(1, IN), f1w_ref[...],
                             (((1,), (1,)), ((), ())),
                             preferred_element_type=jnp.float32)
        p_all = lax.dot_general(e_sel.reshape(B * T, KEY), m_fused,
                                (((1,), (0,)), ((), ())),
                                preferred_element_type=jnp.float32) + cb
        p_all = p_all.reshape(B, T, FUNC)

        x0 = lax.dot_general(ar_ref[...], f1w_ref[...],
                             (((1,), (1,)), ((), ())),
                             preferred_element_type=jnp.float32) + f1b_ref[...][None, :]
        func_embed = lax.dot_general(utm_ref[...], fw_ref[...],
                                     (((1,), (1,)), ((), ())),
                                     preferred_element_type=jnp.float32)
        func_embed = jnp.maximum(func_embed + fb_ref[...][None, :], 0.0)

        bias = (bih_ref[...] + bhh_ref[...])[None, :]
        xi = x0
        h = jnp.zeros((B, HID), dtype=jnp.float32)
        c = jnp.zeros((B, HID), dtype=jnp.float32)
        cnt = jnp.zeros((B, N), dtype=jnp.float32)
        h_rows = []
        pen_rows = []
        for t in range(T + 1):
            r = jnp.maximum(xi + func_embed, 0.0)
            li = lax.dot_general(r, f2w_ref[...], (((1,), (1,)), ((), ())),
                                 preferred_element_type=jnp.float32) + f2b_ref[...][None, :]
            g = lax.dot_general(li, wih_ref[...], (((1,), (1,)), ((), ())),
                                preferred_element_type=jnp.float32)
            g = g + lax.dot_general(h, whh_ref[...], (((1,), (1,)), ((), ())),
                                    preferred_element_type=jnp.float32) + bias
            gi = g[:, 0:HID]
            gf = g[:, HID:2 * HID]
            gg = g[:, 2 * HID:3 * HID]
            go = g[:, 3 * HID:4 * HID]
            c = jax.nn.sigmoid(gf) * c + jax.nn.sigmoid(gi) * jnp.tanh(gg)
            h = jax.nn.sigmoid(go) * jnp.tanh(c)
            h_rows.append(h.reshape(B, 1, HID))
            pen_rows.append((jnp.minimum(cnt, 1.0) * (-1e9)).reshape(B, 1, N))
            if t < T:
                xi = xi + p_all[:, t, :].reshape(B, FUNC)
                cnt = cnt + oh[:, t, :].reshape(B, N)

        h_all = jnp.concatenate(h_rows, axis=1)
        pen = jnp.concatenate(pen_rows, axis=1)
        q = lax.dot_general(h_all, key, (((2,), (2,)), ((0,), (0,))),
                            preferred_element_type=jnp.float32)
        out_ref[:, :, :N] = q + pen
        q_end = jnp.sum(h_all * endw_ref[...][0][None, None, :], axis=2,
                        keepdims=True)
        out_ref[:, :, N:] = q_end


@jax.jit
def _run(autoregressive_embedding, unit_type_mask, entity_embedding,
         selected_units, key_fc_w, key_fc_b, func_fc_w, func_fc_b,
         fc1_w, fc1_b, fc2_w, fc2_b, embed_fc_w, embed_fc_b,
         end_embedding, lstm_w_ih, lstm_w_hh, lstm_b_ih, lstm_b_hh):
    full = lambda a: pl.BlockSpec(a.shape, lambda i: (0,) * a.ndim)
    args = (autoregressive_embedding, unit_type_mask, entity_embedding,
            selected_units, key_fc_w, key_fc_b, func_fc_w, func_fc_b,
            fc1_w, fc1_b, fc2_w, fc2_b, embed_fc_w, embed_fc_b,
            end_embedding, lstm_w_ih, lstm_w_hh, lstm_b_ih, lstm_b_hh)
    in_specs = [full(a) for a in args]
    in_specs[2] = pl.BlockSpec((B, NBLK, entity_embedding.shape[2]),
                               lambda i: (0, i, 0))
    return pl.pallas_call(
        _fused_kernel,
        grid=(GRID,),
        in_specs=in_specs,
        out_specs=pl.BlockSpec((B, T + 1, N + 1), lambda i: (0, 0, 0)),
        out_shape=jax.ShapeDtypeStruct((B, T + 1, N + 1), jnp.float32),
        scratch_shapes=[pltpu.VMEM((B, N, KEY), jnp.float32)],
    )(*args)


def kernel(autoregressive_embedding, unit_type_mask, entity_embedding,
           entity_mask, selected_units, key_fc_w, key_fc_b, func_fc_w,
           func_fc_b, fc1_w, fc1_b, fc2_w, fc2_b, embed_fc_w, embed_fc_b,
           end_embedding, lstm_w_ih, lstm_w_hh, lstm_b_ih, lstm_b_hh):
    return _run(autoregressive_embedding, unit_type_mask, entity_embedding,
                selected_units, key_fc_w, key_fc_b, func_fc_w, func_fc_b,
                fc1_w, fc1_b, fc2_w, fc2_b, embed_fc_w, embed_fc_b,
                end_embedding, lstm_w_ih, lstm_w_hh, lstm_b_ih, lstm_b_hh)


# speedup vs baseline: 6.0973x; 1.0004x over previous
import functools

import jax
import jax.numpy as jnp
from jax import lax
from jax.experimental import pallas as pl
from jax.experimental.pallas import tpu as pltpu

B = 64
N = 512
T = 16
KEY = 32
HID = 32
FUNC = 256
IN = 1024
UT = 259
BBLK = 8
GRID = B // BBLK


def _fused_kernel(ar_ref, utm_ref, ent_ref, sel_ref, kw_ref, kb_ref,
                  fw_ref, fb_ref, f1w_ref, f1b_ref, f2w_ref, f2b_ref,
                  ew_ref, eb_ref, endw_ref, wih_ref, whh_ref, bih_ref,
                  bhh_ref, out_ref, key_s):
    i = pl.program_id(0)

    ent2 = ent_ref[...].reshape(BBLK * N, -1)
    kblk = lax.dot_general(ent2, kw_ref[...], (((1,), (1,)), ((), ())),
                           preferred_element_type=jnp.float32)
    kblk = kblk + kb_ref[...][None, :]
    key_s[pl.ds(i * BBLK, BBLK), :, :] = kblk.reshape(BBLK, N, KEY)

    @pl.when(i == GRID - 1)
    def _decode():
        key = key_s[...]
        sel = sel_ref[...]
        iota_n = lax.broadcasted_iota(jnp.int32, (B, T, N), 2)
        oh = (sel[:, :, None] == iota_n).astype(jnp.float32)

        e_sel = lax.dot_general(oh, key, (((2,), (1,)), ((0,), (0,))),
                                preferred_element_type=jnp.float32)
        e_sel = e_sel * (1.0 / N)

        m_fused = lax.dot_general(ew_ref[...], f1w_ref[...],
                                  (((0,), (1,)), ((), ())),
                                  preferred_element_type=jnp.float32)
        cb = lax.dot_general(eb_ref[...].reshape(1, IN), f1w_ref[...],
                             (((1,), (1,)), ((), ())),
                             preferred_element_type=jnp.float32)
        p_all = lax.dot_general(e_sel.reshape(B * T, KEY), m_fused,
                                (((1,), (0,)), ((), ())),
                                preferred_element_type=jnp.float32) + cb
        p_all = p_all.reshape(B, T, FUNC)

        x0 = lax.dot_general(ar_ref[...], f1w_ref[...],
                             (((1,), (1,)), ((), ())),
                             preferred_element_type=jnp.float32) + f1b_ref[...][None, :]
        func_embed = lax.dot_general(utm_ref[...], fw_ref[...],
                                     (((1,), (1,)), ((), ())),
                                     preferred_element_type=jnp.float32)
        func_embed = jnp.maximum(func_embed + fb_ref[...][None, :], 0.0)

        bias = (bih_ref[...] + bhh_ref[...])[None, :]
        xi = x0
        h = jnp.zeros((B, HID), dtype=jnp.float32)
        c = jnp.zeros((B, HID), dtype=jnp.float32)
        cnt = jnp.zeros((B, N), dtype=jnp.float32)
        h_rows = []
        pen_rows = []
        for t in range(T + 1):
            r = jnp.maximum(xi + func_embed, 0.0)
            li = lax.dot_general(r, f2w_ref[...], (((1,), (1,)), ((), ())),
                                 preferred_element_type=jnp.float32) + f2b_ref[...][None, :]
            g = lax.dot_general(li, wih_ref[...], (((1,), (1,)), ((), ())),
                                preferred_element_type=jnp.float32)
            g = g + lax.dot_general(h, whh_ref[...], (((1,), (1,)), ((), ())),
                                    preferred_element_type=jnp.float32) + bias
            gi = g[:, 0:HID]
            gf = g[:, HID:2 * HID]
            gg = g[:, 2 * HID:3 * HID]
            go = g[:, 3 * HID:4 * HID]
            c = jax.nn.sigmoid(gf) * c + jax.nn.sigmoid(gi) * jnp.tanh(gg)
            h = jax.nn.sigmoid(go) * jnp.tanh(c)
            h_rows.append(h.reshape(B, 1, HID))
            pen_rows.append((jnp.minimum(cnt, 1.0) * (-1e9)).reshape(B, 1, N))
            if t < T:
                xi = xi + p_all[:, t, :].reshape(B, FUNC)
                cnt = cnt + oh[:, t, :].reshape(B, N)

        h_all = jnp.concatenate(h_rows, axis=1)
        pen = jnp.concatenate(pen_rows, axis=1)
        q = lax.dot_general(h_all, key, (((2,), (2,)), ((0,), (0,))),
                            preferred_element_type=jnp.float32)
        out_ref[:, :, :N] = q + pen
        q_end = jnp.sum(h_all * endw_ref[...][0][None, None, :], axis=2,
                        keepdims=True)
        out_ref[:, :, N:] = q_end


@jax.jit
def _run(autoregressive_embedding, unit_type_mask, entity_embedding,
         selected_units, key_fc_w, key_fc_b, func_fc_w, func_fc_b,
         fc1_w, fc1_b, fc2_w, fc2_b, embed_fc_w, embed_fc_b,
         end_embedding, lstm_w_ih, lstm_w_hh, lstm_b_ih, lstm_b_hh):
    full = lambda a: pl.BlockSpec(a.shape, lambda i: (0,) * a.ndim)
    args = (autoregressive_embedding, unit_type_mask, entity_embedding,
            selected_units, key_fc_w, key_fc_b, func_fc_w, func_fc_b,
            fc1_w, fc1_b, fc2_w, fc2_b, embed_fc_w, embed_fc_b,
            end_embedding, lstm_w_ih, lstm_w_hh, lstm_b_ih, lstm_b_hh)
    in_specs = [full(a) for a in args]
    in_specs[2] = pl.BlockSpec((BBLK, N, entity_embedding.shape[2]),
                               lambda i: (i, 0, 0))
    return pl.pallas_call(
        _fused_kernel,
        grid=(GRID,),
        in_specs=in_specs,
        out_specs=pl.BlockSpec((B, T + 1, N + 1), lambda i: (0, 0, 0)),
        out_shape=jax.ShapeDtypeStruct((B, T + 1, N + 1), jnp.float32),
        scratch_shapes=[pltpu.VMEM((B, N, KEY), jnp.float32)],
    )(*args)


def kernel(autoregressive_embedding, unit_type_mask, entity_embedding,
           entity_mask, selected_units, key_fc_w, key_fc_b, func_fc_w,
           func_fc_b, fc1_w, fc1_b, fc2_w, fc2_b, embed_fc_w, embed_fc_b,
           end_embedding, lstm_w_ih, lstm_w_hh, lstm_b_ih, lstm_b_hh):
    return _run(autoregressive_embedding, unit_type_mask, entity_embedding,
                selected_units, key_fc_w, key_fc_b, func_fc_w, func_fc_b,
                fc1_w, fc1_b, fc2_w, fc2_b, embed_fc_w, embed_fc_b,
                end_embedding, lstm_w_ih, lstm_w_hh, lstm_b_ih, lstm_b_hh)


# speedup vs baseline: 6.6924x; 1.0976x over previous
import functools

import jax
import jax.numpy as jnp
from jax import lax
from jax.experimental import pallas as pl
from jax.experimental.pallas import tpu as pltpu

B = 64
N = 512
NP = 513
T = 16
KEY = 32
HID = 32
FUNC = 256
IN = 1024
BBLK = 8
GRID = B // BBLK


def _fused_kernel(ar_ref, utm_ref, ent_ref, sel_ref, kw_ref, kb_ref,
                  fw_ref, fb_ref, f1w_ref, f1b_ref, f2w_ref, f2b_ref,
                  ew_ref, eb_ref, endw_ref, wih_ref, whh_ref, bih_ref,
                  bhh_ref, out_ref, key_s, esel_s, fs_s, m_s, cb_s, w2_s,
                  gxb_s, x0_s, fe_s):
    i = pl.program_id(0)

    ent2 = ent_ref[...].reshape(BBLK * N, -1)
    kblk = lax.dot_general(ent2, kw_ref[...], (((1,), (1,)), ((), ())),
                           preferred_element_type=jnp.float32)
    kblk3 = (kblk + kb_ref[...][None, :]).reshape(BBLK, N, KEY)
    key_s[pl.ds(i * BBLK, BBLK), 0:N, :] = kblk3

    selb = sel_ref[pl.ds(i * BBLK, BBLK), :]
    iota_n = lax.broadcasted_iota(jnp.int32, (BBLK, T, NP), 2)
    ohb = selb[:, :, None] == iota_n
    j_iota = lax.broadcasted_iota(jnp.int32, (BBLK, T, NP), 1)
    fs_s[pl.ds(i * BBLK, BBLK), :] = jnp.min(
        jnp.where(ohb, j_iota, T + 1), axis=1)
    ohbf = ohb[:, :, 0:N].astype(jnp.float32)
    eselb = lax.dot_general(ohbf, kblk3, (((2,), (1,)), ((0,), (0,))),
                            preferred_element_type=jnp.float32)
    esel_s[pl.ds(i * BBLK, BBLK), :, :] = eselb * (1.0 / N)

    @pl.when(i == 0)
    def _prep():
        key_s[:, N:NP, :] = jnp.broadcast_to(endw_ref[...][None, :, :],
                                             (B, 1, KEY))
        m_s[...] = lax.dot_general(ew_ref[...], f1w_ref[...],
                                   (((0,), (1,)), ((), ())),
                                   preferred_element_type=jnp.float32)
        cb_s[...] = lax.dot_general(eb_ref[...].reshape(1, IN), f1w_ref[...],
                                    (((1,), (1,)), ((), ())),
                                    preferred_element_type=jnp.float32)
        w2_s[...] = lax.dot_general(wih_ref[...], f2w_ref[...],
                                    (((1,), (0,)), ((), ())),
                                    preferred_element_type=jnp.float32)
        gxb = lax.dot_general(f2b_ref[...].reshape(1, KEY), wih_ref[...],
                              (((1,), (1,)), ((), ())),
                              preferred_element_type=jnp.float32)
        gxb = gxb + (bih_ref[...] + bhh_ref[...])[None, :]
        gxb_s[...] = lax.transpose(gxb, (1, 0))
        x0_s[...] = lax.dot_general(ar_ref[...], f1w_ref[...],
                                    (((1,), (1,)), ((), ())),
                                    preferred_element_type=jnp.float32
                                    ) + f1b_ref[...][None, :]
        fe = lax.dot_general(utm_ref[...], fw_ref[...],
                             (((1,), (1,)), ((), ())),
                             preferred_element_type=jnp.float32)
        fe_s[...] = jnp.maximum(fe + fb_ref[...][None, :], 0.0)

    @pl.when(i == GRID - 1)
    def _decode():
        key = key_s[...]
        p_all = lax.dot_general(esel_s[...].reshape(B * T, KEY), m_s[...],
                                (((1,), (0,)), ((), ())),
                                preferred_element_type=jnp.float32) + cb_s[...]
        p_all = p_all.reshape(B, T, FUNC)

        fe = fe_s[...]
        xi = x0_s[...]
        r_rows = [jnp.maximum(xi + fe, 0.0)]
        for t in range(T):
            xi = xi + p_all[:, t, :].reshape(B, FUNC)
            r_rows.append(jnp.maximum(xi + fe, 0.0))

        w2 = w2_s[...]
        gxb_col = gxb_s[...]
        gx_list = [
            lax.dot_general(w2, r, (((1,), (1,)), ((), ())),
                            preferred_element_type=jnp.float32) + gxb_col
            for r in r_rows
        ]

        h_t = jnp.zeros((HID, B), dtype=jnp.float32)
        c_t = jnp.zeros((HID, B), dtype=jnp.float32)
        h_rows = []
        for t in range(T + 1):
            g = gx_list[t] + lax.dot_general(
                whh_ref[...], h_t, (((1,), (0,)), ((), ())),
                preferred_element_type=jnp.float32)
            gi = g[0:HID, :]
            gf = g[HID:2 * HID, :]
            gg = g[2 * HID:3 * HID, :]
            go = g[3 * HID:4 * HID, :]
            c_t = jax.nn.sigmoid(gf) * c_t + jax.nn.sigmoid(gi) * jnp.tanh(gg)
            h_t = jax.nn.sigmoid(go) * jnp.tanh(c_t)
            h_rows.append(lax.transpose(h_t, (1, 0)))

        h_all = jnp.concatenate(h_rows, axis=0).reshape(T + 1, B, HID)
        q = lax.dot_general(h_all, key, (((2,), (2,)), ((1,), (0,))),
                            preferred_element_type=jnp.float32)
        t_iota = lax.broadcasted_iota(jnp.int32, (B, T + 1, NP), 1)
        pen = jnp.where(t_iota > fs_s[...][:, None, :], -1e9, 0.0)
        out_ref[...] = q + pen


@jax.jit
def _run(autoregressive_embedding, unit_type_mask, entity_embedding,
         selected_units, key_fc_w, key_fc_b, func_fc_w, func_fc_b,
         fc1_w, fc1_b, fc2_w, fc2_b, embed_fc_w, embed_fc_b,
         end_embedding, lstm_w_ih, lstm_w_hh, lstm_b_ih, lstm_b_hh):
    full = lambda a: pl.BlockSpec(a.shape, lambda i: (0,) * a.ndim)
    args = (autoregressive_embedding, unit_type_mask, entity_embedding,
            selected_units, key_fc_w, key_fc_b, func_fc_w, func_fc_b,
            fc1_w, fc1_b, fc2_w, fc2_b, embed_fc_w, embed_fc_b,
            end_embedding, lstm_w_ih, lstm_w_hh, lstm_b_ih, lstm_b_hh)
    in_specs = [full(a) for a in args]
    in_specs[2] = pl.BlockSpec((BBLK, N, entity_embedding.shape[2]),
                               lambda i: (i, 0, 0))
    return pl.pallas_call(
        _fused_kernel,
        grid=(GRID,),
        in_specs=in_specs,
        out_specs=pl.BlockSpec((B, T + 1, NP), lambda i: (0, 0, 0)),
        out_shape=jax.ShapeDtypeStruct((B, T + 1, NP), jnp.float32),
        scratch_shapes=[
            pltpu.VMEM((B, NP, KEY), jnp.float32),
            pltpu.VMEM((B, T, KEY), jnp.float32),
            pltpu.VMEM((B, NP), jnp.int32),
            pltpu.VMEM((KEY, FUNC), jnp.float32),
            pltpu.VMEM((1, FUNC), jnp.float32),
            pltpu.VMEM((4 * HID, FUNC), jnp.float32),
            pltpu.VMEM((4 * HID, 1), jnp.float32),
            pltpu.VMEM((B, FUNC), jnp.float32),
            pltpu.VMEM((B, FUNC), jnp.float32),
        ],
    )(*args)


def kernel(autoregressive_embedding, unit_type_mask, entity_embedding,
           entity_mask, selected_units, key_fc_w, key_fc_b, func_fc_w,
           func_fc_b, fc1_w, fc1_b, fc2_w, fc2_b, embed_fc_w, embed_fc_b,
           end_embedding, lstm_w_ih, lstm_w_hh, lstm_b_ih, lstm_b_hh):
    return _run(autoregressive_embedding, unit_type_mask, entity_embedding,
                selected_units, key_fc_w, key_fc_b, func_fc_w, func_fc_b,
                fc1_w, fc1_b, fc2_w, fc2_b, embed_fc_w, embed_fc_b,
                end_embedding, lstm_w_ih, lstm_w_hh, lstm_b_ih, lstm_b_hh)
